# Initial kernel scaffold; baseline (speedup 1.0000x reference)
import jax
import jax.numpy as jnp
from jax.experimental import pallas as pl


def kernel(x, edge_index, W, b):
    raise NotImplementedError("write your pallas kernel here")


# speedup vs baseline: 17.3562x; 17.3562x over previous
import functools

import jax
import jax.numpy as jnp
from jax import lax
from jax.experimental import pallas as pl
from jax.experimental.pallas import tpu as pltpu
from jax.experimental.pallas import tpu_sc as plsc

N = 10000
E = 160000
D = 256
HALF = 128
NPAD = 10240
L = 16
NTILES = 16
NCORES = 2

CNT_EDGES = E // 32
CNT_ROWS = (CNT_EDGES + 127) // 128
MAIN_EDGES = E // NTILES
MAIN_ROWS = (MAIN_EDGES + 127) // 128

_mesh = plsc.VectorSubcoreMesh(core_axis_name="c", subcore_axis_name="s")


@functools.partial(
    pl.kernel,
    mesh=_mesh,
    out_type=jax.ShapeDtypeStruct((NCORES, NPAD), jnp.float32),
    scratch_types=[
        pltpu.VMEM((CNT_ROWS, 128), jnp.int32),
        pltpu.VMEM((128,), jnp.float32),
        pltpu.VMEM((640,), jnp.float32),
        pltpu.VMEM_SHARED((NPAD,), jnp.float32),
    ],
)
def _count_kernel(dstp_hbm, cnt_hbm, idx_v, ones_v, buf_v, cnt_sh):
    cid = lax.axis_index("c")
    sid = lax.axis_index("s")
    tile = cid * NTILES + sid

    pltpu.sync_copy(dstp_hbm.at[tile], idx_v)

    def _fill(i, _):
        ones_v[pl.ds(i * L, L)] = jnp.full((L,), 1.0, jnp.float32)
        buf_v[pl.ds(i * L, L)] = jnp.zeros((L,), jnp.float32)
        return 0
    lax.fori_loop(0, 128 // L, _fill, 0)

    def _zfill(i, _):
        buf_v[pl.ds(L * 8 + i * L, L)] = jnp.zeros((L,), jnp.float32)
        return 0
    lax.fori_loop(0, (640 - 128) // L, _zfill, 0)
    pltpu.sync_copy(buf_v, cnt_sh.at[pl.ds(sid * 640, 640)])
    plsc.subcore_barrier()

    def _scat(j, _):
        pltpu.sync_copy(ones_v, cnt_sh.at[idx_v.at[j]], add=True)
        return 0
    lax.fori_loop(0, CNT_ROWS, _scat, 0)
    plsc.subcore_barrier()

    pltpu.sync_copy(cnt_sh.at[pl.ds(sid * 640, 640)], buf_v)
    pltpu.sync_copy(buf_v, cnt_hbm.at[cid, pl.ds(sid * 640, 640)])


def _scale_body(x_ref, w_ref, cnt_ref, y0_ref, y1_ref, dinv_ref):
    deg = cnt_ref[0, :] + cnt_ref[1, :] + 1.0
    dinv = lax.rsqrt(deg)
    xw = jnp.dot(x_ref[...], w_ref[...], preferred_element_type=jnp.float32)
    y = dinv[:, None] * xw
    y0_ref[...] = y[:, :HALF]
    y1_ref[...] = y[:, HALF:]
    dinv_ref[...] = dinv


def _matmul_scale(xp, W, cnt):
    blk = 1024
    grid = (NPAD // blk,)
    return pl.pallas_call(
        _scale_body,
        grid=grid,
        in_specs=[
            pl.BlockSpec((blk, D), lambda i: (i, 0)),
            pl.BlockSpec((D, D), lambda i: (0, 0)),
            pl.BlockSpec((NCORES, blk), lambda i: (0, i)),
        ],
        out_specs=[
            pl.BlockSpec((blk, HALF), lambda i: (i, 0)),
            pl.BlockSpec((blk, HALF), lambda i: (i, 0)),
            pl.BlockSpec((blk,), lambda i: (i,)),
        ],
        out_shape=[
            jax.ShapeDtypeStruct((NPAD, HALF), jnp.float32),
            jax.ShapeDtypeStruct((NPAD, HALF), jnp.float32),
            jax.ShapeDtypeStruct((NPAD,), jnp.float32),
        ],
    )(xp, W, cnt)


@functools.partial(
    pl.kernel,
    mesh=_mesh,
    out_type=jax.ShapeDtypeStruct((NCORES, NPAD, HALF), jnp.float32),
    scratch_types=[
        pltpu.VMEM((MAIN_ROWS, 128), jnp.int32),
        pltpu.VMEM((MAIN_ROWS, 128), jnp.int32),
        pltpu.VMEM((128, HALF), jnp.float32),
        pltpu.VMEM_SHARED((NPAD, HALF), jnp.float32),
    ],
)
def _edge_kernel(srcp_hbm, dstp_hbm, y0_hbm, y1_hbm, acc_hbm,
                 idxs_v, idxd_v, rows_v, acc_sh):
    cid = lax.axis_index("c")
    sid = lax.axis_index("s")

    pltpu.sync_copy(srcp_hbm.at[sid], idxs_v)
    pltpu.sync_copy(dstp_hbm.at[sid], idxd_v)

    def _zrow(i, _):
        def _zlane(j, _):
            rows_v[i, pl.ds(j * L, L)] = jnp.zeros((L,), jnp.float32)
            return 0
        lax.fori_loop(0, HALF // L, _zlane, 0)
        return 0
    lax.fori_loop(0, 128, _zrow, 0)

    def _zacc(k, _):
        pltpu.sync_copy(rows_v, acc_sh.at[pl.ds(sid * 640 + k * 128, 128)])
        return 0
    lax.fori_loop(0, 5, _zacc, 0)
    plsc.subcore_barrier()

    def _sweep(y_hbm):
        def _chunk(j, _):
            pltpu.sync_copy(y_hbm.at[idxs_v.at[j]], rows_v)
            pltpu.sync_copy(rows_v, acc_sh.at[idxd_v.at[j]], add=True)
            return 0
        lax.fori_loop(0, MAIN_ROWS, _chunk, 0)

    @pl.when(cid == 0)
    def _():
        _sweep(y0_hbm)

    @pl.when(cid == 1)
    def _():
        _sweep(y1_hbm)

    plsc.subcore_barrier()

    def _dump(k, _):
        r0 = sid * 640 + k * 128
        pltpu.sync_copy(acc_sh.at[pl.ds(r0, 128)], rows_v)
        pltpu.sync_copy(rows_v, acc_hbm.at[cid, pl.ds(r0, 128)])
        return 0
    lax.fori_loop(0, 5, _dump, 0)


def _final_body(acc_ref, y0_ref, y1_ref, dinv_ref, b_ref, o0_ref, o1_ref):
    dinv = dinv_ref[...]
    o0_ref[...] = dinv[:, None] * (acc_ref[0] + y0_ref[...]) + b_ref[0, :]
    o1_ref[...] = dinv[:, None] * (acc_ref[1] + y1_ref[...]) + b_ref[1, :]


def _finalize(acc, y0, y1, dinv, b2):
    blk = 1024
    return pl.pallas_call(
        _final_body,
        grid=(NPAD // blk,),
        in_specs=[
            pl.BlockSpec((NCORES, blk, HALF), lambda i: (0, i, 0)),
            pl.BlockSpec((blk, HALF), lambda i: (i, 0)),
            pl.BlockSpec((blk, HALF), lambda i: (i, 0)),
            pl.BlockSpec((blk,), lambda i: (i,)),
            pl.BlockSpec((NCORES, HALF), lambda i: (0, 0)),
        ],
        out_specs=[
            pl.BlockSpec((blk, HALF), lambda i: (i, 0)),
            pl.BlockSpec((blk, HALF), lambda i: (i, 0)),
        ],
        out_shape=[
            jax.ShapeDtypeStruct((NPAD, HALF), jnp.float32),
            jax.ShapeDtypeStruct((NPAD, HALF), jnp.float32),
        ],
    )(acc, y0, y1, dinv, b2)


def _pad_idx(flat, ntiles, nrows):
    per = E // ntiles
    slots = nrows * 128
    flat2 = flat.reshape(ntiles, per)
    npad = slots - per
    pad = N + (jnp.arange(ntiles * npad, dtype=jnp.int32) % (NPAD - N))
    pad = pad.reshape(ntiles, npad)
    return jnp.concatenate([flat2, pad], axis=1).reshape(ntiles, nrows, 128)


def kernel(x, edge_index, W, b):
    src = edge_index[0]
    dst = edge_index[1]

    dst1 = _pad_idx(dst, 32, CNT_ROWS)
    src3 = _pad_idx(src, NTILES, MAIN_ROWS)
    dst3 = _pad_idx(dst, NTILES, MAIN_ROWS)
    xp = jnp.pad(x, ((0, NPAD - N), (0, 0)))
    b2 = b.reshape(NCORES, HALF)

    cnt = _count_kernel(dst1)
    y0, y1, dinv = _matmul_scale(xp, W, cnt)
    acc = _edge_kernel(src3, dst3, y0, y1)
    o0, o1 = _finalize(acc, y0, y1, dinv, b2)
    return jnp.concatenate([o0[:N], o1[:N]], axis=1)


# speedup vs baseline: 23.8547x; 1.3744x over previous
import functools

import jax
import jax.numpy as jnp
from jax import lax
from jax.experimental import pallas as pl
from jax.experimental.pallas import tpu as pltpu
from jax.experimental.pallas import tpu_sc as plsc

N = 10000
E = 160000
D = 256
HALF = 128
NPAD = 10240
L = 16
NTILES = 16
NCORES = 2

CNT_EDGES = E // 32
CNT_ROWS = (CNT_EDGES + 127) // 128
MAIN_CHUNK = 128
MAIN_PASSES = 2

_mesh = plsc.VectorSubcoreMesh(core_axis_name="c", subcore_axis_name="s")


@functools.partial(
    pl.kernel,
    mesh=_mesh,
    out_type=jax.ShapeDtypeStruct((NCORES, NPAD), jnp.float32),
    scratch_types=[
        pltpu.VMEM((CNT_ROWS, 128), jnp.int32),
        pltpu.VMEM((128,), jnp.float32),
        pltpu.VMEM((640,), jnp.float32),
        pltpu.VMEM_SHARED((NPAD,), jnp.float32),
    ],
)
def _count_kernel(dstp_hbm, cnt_hbm, idx_v, ones_v, buf_v, cnt_sh):
    cid = lax.axis_index("c")
    sid = lax.axis_index("s")
    tile = cid * NTILES + sid

    pltpu.sync_copy(dstp_hbm.at[tile], idx_v)

    def _fill(i, _):
        ones_v[pl.ds(i * L, L)] = jnp.full((L,), 1.0, jnp.float32)
        buf_v[pl.ds(i * L, L)] = jnp.zeros((L,), jnp.float32)
        return 0
    lax.fori_loop(0, 128 // L, _fill, 0)

    def _zfill(i, _):
        buf_v[pl.ds(L * 8 + i * L, L)] = jnp.zeros((L,), jnp.float32)
        return 0
    lax.fori_loop(0, (640 - 128) // L, _zfill, 0)
    pltpu.sync_copy(buf_v, cnt_sh.at[pl.ds(sid * 640, 640)])
    plsc.subcore_barrier()

    def _scat(j, _):
        pltpu.sync_copy(ones_v, cnt_sh.at[idx_v.at[j]], add=True)
        return 0
    lax.fori_loop(0, CNT_ROWS, _scat, 0)
    plsc.subcore_barrier()

    pltpu.sync_copy(cnt_sh.at[pl.ds(sid * 640, 640)], buf_v)
    pltpu.sync_copy(buf_v, cnt_hbm.at[cid, pl.ds(sid * 640, 640)])


def _scale_body(x_ref, w_ref, cnt_ref, y0_ref, y1_ref, dinv_ref):
    deg = cnt_ref[0, :] + cnt_ref[1, :] + 1.0
    dinv = lax.rsqrt(deg)
    xw = jnp.dot(x_ref[...], w_ref[...], preferred_element_type=jnp.float32)
    y = dinv[:, None] * xw
    y0_ref[...] = y[:, :HALF]
    y1_ref[...] = y[:, HALF:]
    dinv_ref[...] = dinv


def _matmul_scale(xp, W, cnt):
    blk = 1024
    grid = (NPAD // blk,)
    return pl.pallas_call(
        _scale_body,
        grid=grid,
        in_specs=[
            pl.BlockSpec((blk, D), lambda i: (i, 0)),
            pl.BlockSpec((D, D), lambda i: (0, 0)),
            pl.BlockSpec((NCORES, blk), lambda i: (0, i)),
        ],
        out_specs=[
            pl.BlockSpec((blk, HALF), lambda i: (i, 0)),
            pl.BlockSpec((blk, HALF), lambda i: (i, 0)),
            pl.BlockSpec((blk,), lambda i: (i,)),
        ],
        out_shape=[
            jax.ShapeDtypeStruct((NPAD, HALF), jnp.float32),
            jax.ShapeDtypeStruct((NPAD, HALF), jnp.float32),
            jax.ShapeDtypeStruct((NPAD,), jnp.float32),
        ],
    )(xp, W, cnt)


@functools.partial(
    pl.kernel,
    mesh=_mesh,
    out_type=jax.ShapeDtypeStruct((NCORES, NPAD, HALF), jnp.float32),
    scratch_types=[
        pltpu.VMEM((CNT_ROWS, 128), jnp.int32),
        pltpu.VMEM((CNT_ROWS, 128), jnp.int32),
        pltpu.VMEM((MAIN_CHUNK, HALF), jnp.float32),
        pltpu.VMEM((MAIN_CHUNK, HALF), jnp.float32),
        pltpu.VMEM_SHARED((NPAD, HALF), jnp.float32),
        pltpu.SemaphoreType.DMA,
        pltpu.SemaphoreType.DMA,
    ],
)
def _edge_kernel(srcp_hbm, dstp_hbm, y0_hbm, y1_hbm, acc_hbm,
                 idxs_v, idxd_v, rows_a, rows_b, acc_sh, sem_a, sem_b):
    cid = lax.axis_index("c")
    sid = lax.axis_index("s")

    def _zrow(i, _):
        def _zlane(j, _):
            rows_a[i, pl.ds(j * L, L)] = jnp.zeros((L,), jnp.float32)
            return 0
        lax.fori_loop(0, HALF // L, _zlane, 0)
        return 0
    lax.fori_loop(0, MAIN_CHUNK, _zrow, 0)

    def _zacc(k, _):
        pltpu.sync_copy(rows_a, acc_sh.at[pl.ds(sid * 640 + k * 128, 128)])
        return 0
    lax.fori_loop(0, 5, _zacc, 0)
    plsc.subcore_barrier()

    def _sweep_pass(y_hbm, p):
        blk = 2 * sid + p
        pltpu.sync_copy(srcp_hbm.at[blk], idxs_v)
        pltpu.sync_copy(dstp_hbm.at[blk], idxd_v)

        def _gather(j, buf, sem):
            pltpu.async_copy(y_hbm.at[idxs_v.at[j]], buf, sem)

        def _wait(j, buf, sem):
            pltpu.make_async_copy(y_hbm.at[idxs_v.at[j]], buf, sem).wait()

        def _scat(j, buf):
            pltpu.sync_copy(buf, acc_sh.at[idxd_v.at[j]], add=True)

        _gather(0, rows_a, sem_a)

        def _pair(k, _):
            _gather(2 * k + 1, rows_b, sem_b)
            _wait(2 * k, rows_a, sem_a)
            _scat(2 * k, rows_a)
            _gather(2 * k + 2, rows_a, sem_a)
            _wait(2 * k + 1, rows_b, sem_b)
            _scat(2 * k + 1, rows_b)
            return 0
        lax.fori_loop(0, CNT_ROWS // 2 - 1, _pair, 0)

        last = CNT_ROWS - 1
        _gather(last, rows_b, sem_b)
        _wait(last - 1, rows_a, sem_a)
        _scat(last - 1, rows_a)
        _wait(last, rows_b, sem_b)
        _scat(last, rows_b)

    @pl.when(cid == 0)
    def _():
        for p in range(MAIN_PASSES):
            _sweep_pass(y0_hbm, p)

    @pl.when(cid == 1)
    def _():
        for p in range(MAIN_PASSES):
            _sweep_pass(y1_hbm, p)

    plsc.subcore_barrier()

    def _dump(k, _):
        r0 = sid * 640 + k * 128
        pltpu.sync_copy(acc_sh.at[pl.ds(r0, 128)], rows_a)
        pltpu.sync_copy(rows_a, acc_hbm.at[cid, pl.ds(r0, 128)])
        return 0
    lax.fori_loop(0, 5, _dump, 0)


def _final_body(acc_ref, y0_ref, y1_ref, dinv_ref, b_ref, o0_ref, o1_ref):
    dinv = dinv_ref[...]
    o0_ref[...] = dinv[:, None] * (acc_ref[0] + y0_ref[...]) + b_ref[0, :]
    o1_ref[...] = dinv[:, None] * (acc_ref[1] + y1_ref[...]) + b_ref[1, :]


def _finalize(acc, y0, y1, dinv, b2):
    blk = 1024
    return pl.pallas_call(
        _final_body,
        grid=(NPAD // blk,),
        in_specs=[
            pl.BlockSpec((NCORES, blk, HALF), lambda i: (0, i, 0)),
            pl.BlockSpec((blk, HALF), lambda i: (i, 0)),
            pl.BlockSpec((blk, HALF), lambda i: (i, 0)),
            pl.BlockSpec((blk,), lambda i: (i,)),
            pl.BlockSpec((NCORES, HALF), lambda i: (0, 0)),
        ],
        out_specs=[
            pl.BlockSpec((blk, HALF), lambda i: (i, 0)),
            pl.BlockSpec((blk, HALF), lambda i: (i, 0)),
        ],
        out_shape=[
            jax.ShapeDtypeStruct((NPAD, HALF), jnp.float32),
            jax.ShapeDtypeStruct((NPAD, HALF), jnp.float32),
        ],
    )(acc, y0, y1, dinv, b2)


def _pad_idx(flat, ntiles, nrows, width):
    per = E // ntiles
    slots = nrows * width
    flat2 = flat.reshape(ntiles, per)
    npad = slots - per
    pad = N + (jnp.arange(ntiles * npad, dtype=jnp.int32) % (NPAD - N))
    pad = pad.reshape(ntiles, npad)
    return jnp.concatenate([flat2, pad], axis=1).reshape(ntiles, nrows, width)


def kernel(x, edge_index, W, b):
    src = edge_index[0]
    dst = edge_index[1]

    dst1 = _pad_idx(dst, 32, CNT_ROWS, 128)
    src3 = _pad_idx(src, 32, CNT_ROWS, 128)
    dst3 = dst1
    xp = jnp.pad(x, ((0, NPAD - N), (0, 0)))
    b2 = b.reshape(NCORES, HALF)

    cnt = _count_kernel(dst1)
    y0, y1, dinv = _matmul_scale(xp, W, cnt)
    acc = _edge_kernel(src3, dst3, y0, y1)
    o0, o1 = _finalize(acc, y0, y1, dinv, b2)
    return jnp.concatenate([o0[:N], o1[:N]], axis=1)


# speedup vs baseline: 24.3025x; 1.0188x over previous
import functools

import jax
import jax.numpy as jnp
from jax import lax
from jax.experimental import pallas as pl
from jax.experimental.pallas import tpu as pltpu
from jax.experimental.pallas import tpu_sc as plsc

N = 10000
E = 160000
D = 256
HALF = 128
NPAD = 10240
L = 16
NTILES = 16
NCORES = 2

CNT_EDGES = E // 32
CNT_ROWS = (CNT_EDGES + 127) // 128
MAIN_CHUNK = 128
MAIN_PASSES = 2

_mesh = plsc.VectorSubcoreMesh(core_axis_name="c", subcore_axis_name="s")


@functools.partial(
    pl.kernel,
    mesh=_mesh,
    out_type=jax.ShapeDtypeStruct((NCORES, NPAD), jnp.float32),
    scratch_types=[
        pltpu.VMEM((CNT_ROWS, 128), jnp.int32),
        pltpu.VMEM((128,), jnp.float32),
        pltpu.VMEM((640,), jnp.float32),
        pltpu.VMEM_SHARED((NPAD,), jnp.float32),
    ],
)
def _count_kernel(dstp_hbm, cnt_hbm, idx_v, ones_v, buf_v, cnt_sh):
    cid = lax.axis_index("c")
    sid = lax.axis_index("s")
    tile = cid * NTILES + sid

    pltpu.sync_copy(dstp_hbm.at[tile], idx_v)

    def _fill(i, _):
        ones_v[pl.ds(i * L, L)] = jnp.full((L,), 1.0, jnp.float32)
        buf_v[pl.ds(i * L, L)] = jnp.zeros((L,), jnp.float32)
        return 0
    lax.fori_loop(0, 128 // L, _fill, 0)

    def _zfill(i, _):
        buf_v[pl.ds(L * 8 + i * L, L)] = jnp.zeros((L,), jnp.float32)
        return 0
    lax.fori_loop(0, (640 - 128) // L, _zfill, 0)
    pltpu.sync_copy(buf_v, cnt_sh.at[pl.ds(sid * 640, 640)])
    plsc.subcore_barrier()

    def _scat(j, _):
        pltpu.sync_copy(ones_v, cnt_sh.at[idx_v.at[j]], add=True)
        return 0
    lax.fori_loop(0, CNT_ROWS, _scat, 0)
    plsc.subcore_barrier()

    pltpu.sync_copy(cnt_sh.at[pl.ds(sid * 640, 640)], buf_v)
    pltpu.sync_copy(buf_v, cnt_hbm.at[cid, pl.ds(sid * 640, 640)])


def _scale_body(x_ref, w_ref, cnt_ref, y0_ref, y1_ref, dinv_ref):
    deg = cnt_ref[:, 0] + cnt_ref[:, 1] + 1.0
    dinv = lax.rsqrt(deg)
    xw = jnp.dot(x_ref[...], w_ref[...], preferred_element_type=jnp.float32)
    y = dinv[:, None] * xw
    y0_ref[...] = y[:, :HALF]
    y1_ref[...] = y[:, HALF:]
    dinv_ref[...] = dinv[:, None]


def _matmul_scale(x, W, cnt):
    blk = 1000
    grid = (N // blk,)
    return pl.pallas_call(
        _scale_body,
        grid=grid,
        in_specs=[
            pl.BlockSpec((blk, D), lambda i: (i, 0)),
            pl.BlockSpec((D, D), lambda i: (0, 0)),
            pl.BlockSpec((blk, NCORES), lambda i: (i, 0)),
        ],
        out_specs=[
            pl.BlockSpec((blk, HALF), lambda i: (i, 0)),
            pl.BlockSpec((blk, HALF), lambda i: (i, 0)),
            pl.BlockSpec((blk, 1), lambda i: (i, 0)),
        ],
        out_shape=[
            jax.ShapeDtypeStruct((N, HALF), jnp.float32),
            jax.ShapeDtypeStruct((N, HALF), jnp.float32),
            jax.ShapeDtypeStruct((N, 1), jnp.float32),
        ],
    )(x, W, cnt)


@functools.partial(
    pl.kernel,
    mesh=_mesh,
    out_type=jax.ShapeDtypeStruct((NCORES, NPAD, HALF), jnp.float32),
    scratch_types=[
        pltpu.VMEM((CNT_ROWS, 128), jnp.int32),
        pltpu.VMEM((CNT_ROWS, 128), jnp.int32),
        pltpu.VMEM((MAIN_CHUNK, HALF), jnp.float32),
        pltpu.VMEM((MAIN_CHUNK, HALF), jnp.float32),
        pltpu.VMEM_SHARED((NPAD, HALF), jnp.float32),
        pltpu.SemaphoreType.DMA,
        pltpu.SemaphoreType.DMA,
    ],
)
def _edge_kernel(srcp_hbm, dstp_hbm, y0_hbm, y1_hbm, acc_hbm,
                 idxs_v, idxd_v, rows_a, rows_b, acc_sh, sem_a, sem_b):
    cid = lax.axis_index("c")
    sid = lax.axis_index("s")

    def _zrow(i, _):
        def _zlane(j, _):
            rows_a[i, pl.ds(j * L, L)] = jnp.zeros((L,), jnp.float32)
            return 0
        lax.fori_loop(0, HALF // L, _zlane, 0)
        return 0
    lax.fori_loop(0, MAIN_CHUNK, _zrow, 0)

    def _zacc(k, _):
        pltpu.sync_copy(rows_a, acc_sh.at[pl.ds(sid * 640 + k * 128, 128)])
        return 0
    lax.fori_loop(0, 5, _zacc, 0)
    plsc.subcore_barrier()

    def _sweep_pass(y_hbm, p):
        blk = 2 * sid + p
        pltpu.sync_copy(srcp_hbm.at[blk], idxs_v)
        pltpu.sync_copy(dstp_hbm.at[blk], idxd_v)

        def _gather(j, buf, sem):
            pltpu.async_copy(y_hbm.at[idxs_v.at[j]], buf, sem)

        def _wait(j, buf, sem):
            pltpu.make_async_copy(y_hbm.at[idxs_v.at[j]], buf, sem).wait()

        def _scat(j, buf):
            pltpu.sync_copy(buf, acc_sh.at[idxd_v.at[j]], add=True)

        _gather(0, rows_a, sem_a)

        def _pair(k, _):
            _gather(2 * k + 1, rows_b, sem_b)
            _wait(2 * k, rows_a, sem_a)
            _scat(2 * k, rows_a)
            _gather(2 * k + 2, rows_a, sem_a)
            _wait(2 * k + 1, rows_b, sem_b)
            _scat(2 * k + 1, rows_b)
            return 0
        lax.fori_loop(0, CNT_ROWS // 2 - 1, _pair, 0)

        last = CNT_ROWS - 1
        _gather(last, rows_b, sem_b)
        _wait(last - 1, rows_a, sem_a)
        _scat(last - 1, rows_a)
        _wait(last, rows_b, sem_b)
        _scat(last, rows_b)

    @pl.when(cid == 0)
    def _():
        for p in range(MAIN_PASSES):
            _sweep_pass(y0_hbm, p)

    @pl.when(cid == 1)
    def _():
        for p in range(MAIN_PASSES):
            _sweep_pass(y1_hbm, p)

    plsc.subcore_barrier()

    def _dump(k, _):
        r0 = sid * 640 + k * 128
        pltpu.sync_copy(acc_sh.at[pl.ds(r0, 128)], rows_a)
        pltpu.sync_copy(rows_a, acc_hbm.at[cid, pl.ds(r0, 128)])
        return 0
    lax.fori_loop(0, 5, _dump, 0)


def _final_body(acc_ref, y0_ref, y1_ref, dinv_ref, b_ref, o_ref):
    dinv = dinv_ref[:, 0]
    o_ref[:, :HALF] = dinv[:, None] * (acc_ref[0] + y0_ref[...]) + b_ref[0, :]
    o_ref[:, HALF:] = dinv[:, None] * (acc_ref[1] + y1_ref[...]) + b_ref[1, :]


def _finalize(acc, y0, y1, dinv, b2):
    blk = 1000
    return pl.pallas_call(
        _final_body,
        grid=(N // blk,),
        in_specs=[
            pl.BlockSpec((NCORES, blk, HALF), lambda i: (0, i, 0)),
            pl.BlockSpec((blk, HALF), lambda i: (i, 0)),
            pl.BlockSpec((blk, HALF), lambda i: (i, 0)),
            pl.BlockSpec((blk, 1), lambda i: (i, 0)),
            pl.BlockSpec((NCORES, HALF), lambda i: (0, 0)),
        ],
        out_specs=pl.BlockSpec((blk, D), lambda i: (i, 0)),
        out_shape=jax.ShapeDtypeStruct((N, D), jnp.float32),
    )(acc, y0, y1, dinv, b2)


def _pad_idx(flat, ntiles, nrows, width, pad_base):
    per = E // ntiles
    slots = nrows * width
    flat2 = flat.reshape(ntiles, per)
    npad = slots - per
    pad = pad_base + (jnp.arange(ntiles * npad, dtype=jnp.int32) % (NPAD - N))
    pad = pad.reshape(ntiles, npad)
    return jnp.concatenate([flat2, pad], axis=1).reshape(ntiles, nrows, width)


def kernel(x, edge_index, W, b):
    src = edge_index[0]
    dst = edge_index[1]

    dst1 = _pad_idx(dst, 32, CNT_ROWS, 128, N)
    src3 = _pad_idx(src, 32, CNT_ROWS, 128, 0)
    dst3 = dst1
    b2 = b.reshape(NCORES, HALF)

    cnt = _count_kernel(dst1)
    y0, y1, dinv = _matmul_scale(x, W, cnt.T)
    acc = _edge_kernel(src3, dst3, y0, y1)
    return _finalize(acc, y0, y1, dinv, b2)


# speedup vs baseline: 25.9051x; 1.0659x over previous
import functools

import jax
import jax.numpy as jnp
from jax import lax
from jax.experimental import pallas as pl
from jax.experimental.pallas import tpu as pltpu
from jax.experimental.pallas import tpu_sc as plsc

N = 10000
E = 160000
D = 256
HALF = 128
NPAD = 10240
L = 16
NTILES = 16
NCORES = 2

CNT_EDGES = E // 32
CNT_ROWS = (CNT_EDGES + 127) // 128
MAIN_CHUNK = 80
MAIN_ROWS = 63
MAIN_PASSES = 2

_mesh = plsc.VectorSubcoreMesh(core_axis_name="c", subcore_axis_name="s")


@functools.partial(
    pl.kernel,
    mesh=_mesh,
    out_type=jax.ShapeDtypeStruct((NCORES, NPAD), jnp.float32),
    scratch_types=[
        pltpu.VMEM((CNT_ROWS, 128), jnp.int32),
        pltpu.VMEM((128,), jnp.float32),
        pltpu.VMEM((640,), jnp.float32),
        pltpu.VMEM_SHARED((NPAD,), jnp.float32),
    ],
)
def _count_kernel(dstp_hbm, cnt_hbm, idx_v, ones_v, buf_v, cnt_sh):
    cid = lax.axis_index("c")
    sid = lax.axis_index("s")
    tile = cid * NTILES + sid

    pltpu.sync_copy(dstp_hbm.at[tile], idx_v)

    def _fill(i, _):
        ones_v[pl.ds(i * L, L)] = jnp.full((L,), 1.0, jnp.float32)
        buf_v[pl.ds(i * L, L)] = jnp.zeros((L,), jnp.float32)
        return 0
    lax.fori_loop(0, 128 // L, _fill, 0)

    def _zfill(i, _):
        buf_v[pl.ds(L * 8 + i * L, L)] = jnp.zeros((L,), jnp.float32)
        return 0
    lax.fori_loop(0, (640 - 128) // L, _zfill, 0)
    pltpu.sync_copy(buf_v, cnt_sh.at[pl.ds(sid * 640, 640)])
    plsc.subcore_barrier()

    def _scat(j, _):
        pltpu.sync_copy(ones_v, cnt_sh.at[idx_v.at[j]], add=True)
        return 0
    lax.fori_loop(0, CNT_ROWS, _scat, 0)
    plsc.subcore_barrier()

    pltpu.sync_copy(cnt_sh.at[pl.ds(sid * 640, 640)], buf_v)
    pltpu.sync_copy(buf_v, cnt_hbm.at[cid, pl.ds(sid * 640, 640)])


def _scale_body(x_ref, w_ref, cnt_ref, y0_ref, y1_ref, dinv_ref):
    deg = cnt_ref[:, 0] + cnt_ref[:, 1] + 1.0
    dinv = lax.rsqrt(deg)
    xw = jnp.dot(x_ref[...], w_ref[...], preferred_element_type=jnp.float32)
    y = dinv[:, None] * xw
    y0_ref[...] = y[:, :HALF]
    y1_ref[...] = y[:, HALF:]
    dinv_ref[...] = dinv[:, None]


def _matmul_scale(x, W, cnt):
    blk = 1000
    grid = (N // blk,)
    return pl.pallas_call(
        _scale_body,
        grid=grid,
        in_specs=[
            pl.BlockSpec((blk, D), lambda i: (i, 0)),
            pl.BlockSpec((D, D), lambda i: (0, 0)),
            pl.BlockSpec((blk, NCORES), lambda i: (i, 0)),
        ],
        out_specs=[
            pl.BlockSpec((blk, HALF), lambda i: (i, 0)),
            pl.BlockSpec((blk, HALF), lambda i: (i, 0)),
            pl.BlockSpec((blk, 1), lambda i: (i, 0)),
        ],
        out_shape=[
            jax.ShapeDtypeStruct((N, HALF), jnp.float32),
            jax.ShapeDtypeStruct((N, HALF), jnp.float32),
            jax.ShapeDtypeStruct((N, 1), jnp.float32),
        ],
    )(x, W, cnt)


@functools.partial(
    pl.kernel,
    mesh=_mesh,
    out_type=jax.ShapeDtypeStruct((NCORES, NPAD, HALF), jnp.float32),
    scratch_types=[
        pltpu.VMEM((MAIN_ROWS, MAIN_CHUNK), jnp.int32),
        pltpu.VMEM((MAIN_ROWS, MAIN_CHUNK), jnp.int32),
        pltpu.VMEM((MAIN_CHUNK, HALF), jnp.float32),
        pltpu.VMEM((MAIN_CHUNK, HALF), jnp.float32),
        pltpu.VMEM((MAIN_CHUNK, HALF), jnp.float32),
        pltpu.VMEM_SHARED((NPAD, HALF), jnp.float32),
        pltpu.SemaphoreType.DMA,
        pltpu.SemaphoreType.DMA,
        pltpu.SemaphoreType.DMA,
        pltpu.SemaphoreType.DMA,
        pltpu.SemaphoreType.DMA,
        pltpu.SemaphoreType.DMA,
    ],
)
def _edge_kernel(srcp_hbm, dstp_hbm, y0_hbm, y1_hbm, acc_hbm,
                 idxs_v, idxd_v, rows_0, rows_1, rows_2, acc_sh,
                 sg_0, sg_1, sg_2, ss_0, ss_1, ss_2):
    cid = lax.axis_index("c")
    sid = lax.axis_index("s")
    bufs = (rows_0, rows_1, rows_2)
    gsems = (sg_0, sg_1, sg_2)
    ssems = (ss_0, ss_1, ss_2)

    def _zrow(i, _):
        def _zlane(j, _):
            rows_0[i, pl.ds(j * L, L)] = jnp.zeros((L,), jnp.float32)
            return 0
        lax.fori_loop(0, HALF // L, _zlane, 0)
        return 0
    lax.fori_loop(0, MAIN_CHUNK, _zrow, 0)

    def _zacc(k, _):
        pltpu.sync_copy(rows_0,
                        acc_sh.at[pl.ds(sid * 640 + k * MAIN_CHUNK,
                                        MAIN_CHUNK)])
        return 0
    lax.fori_loop(0, 640 // MAIN_CHUNK, _zacc, 0)
    plsc.subcore_barrier()

    def _sweep_pass(y_hbm, p):
        blk = 2 * sid + p
        pltpu.sync_copy(srcp_hbm.at[blk], idxs_v)
        pltpu.sync_copy(dstp_hbm.at[blk], idxd_v)

        def _gather(j, bi):
            pltpu.async_copy(y_hbm.at[idxs_v.at[j]], bufs[bi], gsems[bi])

        def _gwait(j, bi):
            pltpu.make_async_copy(
                y_hbm.at[idxs_v.at[j]], bufs[bi], gsems[bi]).wait()

        def _scat(j, bi):
            pltpu.async_copy(bufs[bi], acc_sh.at[idxd_v.at[j]], ssems[bi],
                             add=True)

        def _swait(j, bi):
            pltpu.make_async_copy(
                bufs[bi], acc_sh.at[idxd_v.at[j]], ssems[bi]).wait()

        _gather(0, 0)
        _gather(1, 1)
        _gather(2, 2)
        _gwait(0, 0)
        _scat(0, 0)

        def _step(j, bi):
            _swait(j - 3, bi)
            _gather(j, bi)
            _gwait(j - 2, (bi + 1) % 3)
            _scat(j - 2, (bi + 1) % 3)

        def _trio(k, _):
            _step(3 * k, 0)
            _step(3 * k + 1, 1)
            _step(3 * k + 2, 2)
            return 0
        lax.fori_loop(1, MAIN_ROWS // 3, _trio, 0)

        last = MAIN_ROWS - 1
        _gwait(last - 1, (last - 1) % 3)
        _scat(last - 1, (last - 1) % 3)
        _gwait(last, last % 3)
        _scat(last, last % 3)
        _swait(last - 2, (last - 2) % 3)
        _swait(last - 1, (last - 1) % 3)
        _swait(last, last % 3)

    @pl.when(cid == 0)
    def _():
        for p in range(MAIN_PASSES):
            _sweep_pass(y0_hbm, p)

    @pl.when(cid == 1)
    def _():
        for p in range(MAIN_PASSES):
            _sweep_pass(y1_hbm, p)

    plsc.subcore_barrier()

    def _dump(k, _):
        r0 = sid * 640 + k * MAIN_CHUNK
        pltpu.sync_copy(acc_sh.at[pl.ds(r0, MAIN_CHUNK)], rows_0)
        pltpu.sync_copy(rows_0, acc_hbm.at[cid, pl.ds(r0, MAIN_CHUNK)])
        return 0
    lax.fori_loop(0, 640 // MAIN_CHUNK, _dump, 0)


def _final_body(acc_ref, y0_ref, y1_ref, dinv_ref, b_ref, o_ref):
    dinv = dinv_ref[:, 0]
    o_ref[:, :HALF] = dinv[:, None] * (acc_ref[0] + y0_ref[...]) + b_ref[0, :]
    o_ref[:, HALF:] = dinv[:, None] * (acc_ref[1] + y1_ref[...]) + b_ref[1, :]


def _finalize(acc, y0, y1, dinv, b2):
    blk = 1000
    return pl.pallas_call(
        _final_body,
        grid=(N // blk,),
        in_specs=[
            pl.BlockSpec((NCORES, blk, HALF), lambda i: (0, i, 0)),
            pl.BlockSpec((blk, HALF), lambda i: (i, 0)),
            pl.BlockSpec((blk, HALF), lambda i: (i, 0)),
            pl.BlockSpec((blk, 1), lambda i: (i, 0)),
            pl.BlockSpec((NCORES, HALF), lambda i: (0, 0)),
        ],
        out_specs=pl.BlockSpec((blk, D), lambda i: (i, 0)),
        out_shape=jax.ShapeDtypeStruct((N, D), jnp.float32),
    )(acc, y0, y1, dinv, b2)


def _pad_idx(flat, ntiles, nrows, width, pad_base):
    per = E // ntiles
    slots = nrows * width
    flat2 = flat.reshape(ntiles, per)
    npad = slots - per
    pad = pad_base + (jnp.arange(ntiles * npad, dtype=jnp.int32) % (NPAD - N))
    pad = pad.reshape(ntiles, npad)
    return jnp.concatenate([flat2, pad], axis=1).reshape(ntiles, nrows, width)


def kernel(x, edge_index, W, b):
    src = edge_index[0]
    dst = edge_index[1]

    dst1 = _pad_idx(dst, 32, CNT_ROWS, 128, N)
    src3 = _pad_idx(src, 32, MAIN_ROWS, MAIN_CHUNK, 0)
    dst3 = _pad_idx(dst, 32, MAIN_ROWS, MAIN_CHUNK, N)
    b2 = b.reshape(NCORES, HALF)

    cnt = _count_kernel(dst1)
    y0, y1, dinv = _matmul_scale(x, W, cnt.T)
    acc = _edge_kernel(src3, dst3, y0, y1)
    return _finalize(acc, y0, y1, dinv, b2)
